# baseline (speedup 1.0000x reference)
import functools

import jax
import jax.numpy as jnp
from jax import lax
from jax.experimental import pallas as pl
from jax.experimental.pallas import tpu as pltpu
from jax.experimental.pallas import tpu_sc as plsc

B, N, K = 8, 2048, 20
BN = B * N
EPS = 1e-5
NEG = -3.0e38


def _knn_body(xt_ref, xa_ref, xxc_ref, xxr_ref, idx_ref, *, tn):
    b = pl.program_id(0)
    xt = xt_ref[0]
    xa = xa_ref[0]
    dot = lax.dot_general(xt, xa, (((1,), (1,)), ((), ())))
    pd = (2.0 * dot - xxc_ref[0]) - xxr_ref[0]
    base = b * N
    q = N // 4
    iota = lax.broadcasted_iota(jnp.int32, (tn, q), 1)
    vals = [pd[:, i * q:(i + 1) * q] for i in range(4)]
    idxs = [iota + i * q for i in range(4)]

    def comp(p, r):
        sw = vals[r] > vals[p]
        vp = jnp.where(sw, vals[r], vals[p])
        vr = jnp.where(sw, vals[p], vals[r])
        ip = jnp.where(sw, idxs[r], idxs[p])
        ir = jnp.where(sw, idxs[p], idxs[r])
        vals[p], vals[r], idxs[p], idxs[r] = vp, vr, ip, ir

    for p, r in ((0, 1), (1, 2), (2, 3), (0, 1), (1, 2), (0, 1)):
        comp(p, r)
    s0, s1, s2, s3 = vals
    i0, i1, i2, i3 = idxs
    for it in range(K):
        m = jnp.max(s0, axis=1, keepdims=True)
        cand = jnp.where(s0 == m, i0, N)
        j = jnp.min(cand, axis=1, keepdims=True)
        idx_ref[0, :, pl.ds(it, 1)] = j + base
        hit = i0 == j
        s0 = jnp.where(hit, s1, s0)
        i0 = jnp.where(hit, i1, i0)
        s1 = jnp.where(hit, s2, s1)
        i1 = jnp.where(hit, i2, i1)
        s2 = jnp.where(hit, s3, s2)
        i2 = jnp.where(hit, i3, i2)
        s3 = jnp.where(hit, NEG, s3)


def _knn(xp, xx, tn=256):
    cp = xp.shape[-1]
    return pl.pallas_call(
        functools.partial(_knn_body, tn=tn),
        grid=(B, N // tn),
        in_specs=[
            pl.BlockSpec((1, tn, cp), lambda b, t: (b, t, 0)),
            pl.BlockSpec((1, N, cp), lambda b, t: (b, 0, 0)),
            pl.BlockSpec((1, tn, 1), lambda b, t: (b, t, 0)),
            pl.BlockSpec((1, 1, N), lambda b, t: (b, 0, 0)),
        ],
        out_specs=pl.BlockSpec((1, tn, K), lambda b, t: (b, t, 0)),
        out_shape=jax.ShapeDtypeStruct((B, N, K), jnp.int32),
    )(xp, xp, xx[:, :, None], xx[:, None, :])


_NC, _NS = 2, 16
_NW = _NC * _NS
_PTS = BN // _NW
_P = 32
_NIDX = _P * K // 128
_NCHUNK = _PTS // _P


@functools.lru_cache(maxsize=None)
def _make_sc_diff(c):
    mesh = plsc.VectorSubcoreMesh(core_axis_name="c", subcore_axis_name="s")

    @functools.partial(
        pl.kernel,
        mesh=mesh,
        compiler_params=pltpu.CompilerParams(use_tc_tiling_on_sc=False),
        out_type=jax.ShapeDtypeStruct((BN * K, c), jnp.float32),
        scratch_types=[
            pltpu.VMEM((_PTS * K // 128, 128), jnp.int32),
            pltpu.VMEM((2, _P * K, c), jnp.float32),
            pltpu.VMEM((2, _P, c), jnp.float32),
            pltpu.SemaphoreType.DMA,
            pltpu.SemaphoreType.DMA,
            pltpu.SemaphoreType.DMA,
            pltpu.SemaphoreType.DMA,
            pltpu.SemaphoreType.DMA,
            pltpu.SemaphoreType.DMA,
        ],
    )
    def sc_diff(x_hbm, idx_hbm, diff_hbm, idx_v, rows_v, self_v,
                g0, g1, s0, s1, o0, o1):
        gsem = (g0, g1)
        ssem = (s0, s1)
        osem = (o0, o1)
        cid = lax.axis_index("c")
        sid = lax.axis_index("s")
        wid = cid * _NS + sid
        base = wid * _PTS
        nrow = _PTS * K // 128
        pltpu.sync_copy(idx_hbm.at[pl.ds(wid * nrow, nrow)], idx_v)

        def fire(ch, buf):
            for i in range(_NIDX):
                pltpu.async_copy(x_hbm.at[idx_v.at[ch * _NIDX + i]],
                                 rows_v.at[buf, pl.ds(i * 128, 128)],
                                 gsem[buf])
            pltpu.async_copy(x_hbm.at[pl.ds(base + ch * _P, _P)],
                             self_v.at[buf], ssem[buf])

        def drain(buf):
            for i in range(_NIDX):
                pltpu.make_async_copy(
                    x_hbm.at[pl.ds(0, 128)],
                    rows_v.at[buf, pl.ds(i * 128, 128)], gsem[buf]).wait()
            pltpu.make_async_copy(x_hbm.at[pl.ds(base, _P)],
                                  self_v.at[buf], ssem[buf]).wait()

        def wait_out(buf):
            pltpu.make_async_copy(diff_hbm.at[pl.ds(0, _P * K)],
                                  rows_v.at[buf], osem[buf]).wait()

        def step(ch, buf):
            @pl.when(ch + 1 < _NCHUNK)
            def _():
                @pl.when(ch >= 1)
                def _():
                    wait_out(1 - buf)
                fire(ch + 1, 1 - buf)

            drain(buf)

            def pt_body(p, carry2):
                r = p * K
                for cc in range(c // 16):
                    sl = pl.ds(cc * 16, 16)
                    s = self_v[buf, p, sl]
                    for kk in range(K):
                        rows_v[buf, r + kk, sl] = rows_v[buf, r + kk, sl] - s
                return carry2

            lax.fori_loop(0, _P, pt_body, 0)
            pbase = base + ch * _P
            pltpu.async_copy(rows_v.at[buf],
                             diff_hbm.at[pl.ds(pbase * K, _P * K)],
                             osem[buf])

        fire(0, 0)

        def pair_body(g, carry):
            step(2 * g, 0)
            step(2 * g + 1, 1)
            return carry

        lax.fori_loop(0, _NCHUNK // 2, pair_body, 0)
        wait_out(0)
        wait_out(1)

    return sc_diff


def _conv_body(diff_ref, x_ref, wd_ref, wx_ref, s_ref, b_ref,
               xn_ref, hm_ref, *, c):
    t = pl.program_id(1)
    dd = diff_ref[0]
    acc = lax.dot_general(dd[:, :c], wd_ref[...], (((1,), (0,)), ((), ())))
    for kk in range(1, K):
        acc = jnp.maximum(acc, lax.dot_general(
            dd[:, kk * c:(kk + 1) * c], wd_ref[...],
            (((1,), (0,)), ((), ()))))
    ys = lax.dot_general(x_ref[0], wx_ref[...], (((1,), (0,)), ((), ())))
    y = jnp.maximum(s_ref[0] * (acc + ys) + b_ref[0], 0.0)
    xn_ref[0] = y
    tm = jnp.max(y, axis=0, keepdims=True)

    @pl.when(t == 0)
    def _():
        hm_ref[0] = tm

    @pl.when(t != 0)
    def _():
        hm_ref[0] = jnp.maximum(hm_ref[0], tm)


def _conv(diff, xp, wd, wx, s, bias, o, tn=256):
    c = xp.shape[-1]
    return pl.pallas_call(
        functools.partial(_conv_body, c=c),
        grid=(B, N // tn),
        in_specs=[
            pl.BlockSpec((1, tn, K * c), lambda b, t: (b, t, 0)),
            pl.BlockSpec((1, tn, c), lambda b, t: (b, t, 0)),
            pl.BlockSpec((c, o), lambda b, t: (0, 0)),
            pl.BlockSpec((c, o), lambda b, t: (0, 0)),
            pl.BlockSpec((1, o), lambda b, t: (0, 0)),
            pl.BlockSpec((1, o), lambda b, t: (0, 0)),
        ],
        out_specs=[
            pl.BlockSpec((1, tn, o), lambda b, t: (b, t, 0)),
            pl.BlockSpec((1, 1, o), lambda b, t: (b, 0, 0)),
        ],
        out_shape=[
            jax.ShapeDtypeStruct((B, N, o), jnp.float32),
            jax.ShapeDtypeStruct((B, 1, o), jnp.float32),
        ],
    )(diff, xp, wd, wx, s, bias)


def _head_body(p1_ref, p2_ref, p3_ref, p4_ref, w1_ref, l1b_ref, s1_ref,
               b1_ref, w2_ref, l2b_ref, s2_ref, b2_ref, w3_ref, l3b_ref,
               out_ref):
    h = jnp.concatenate([p1_ref[:, 0], p2_ref[:, 0], p3_ref[:, 0],
                         p4_ref[:, 0]], axis=1)
    t1 = lax.dot_general(h, w1_ref[...], (((1,), (0,)), ((), ()))) + l1b_ref[0]
    h1 = jax.nn.relu(t1 * s1_ref[0] + b1_ref[0])
    t2 = lax.dot_general(h1, w2_ref[...], (((1,), (0,)), ((), ()))) \
        + l2b_ref[0]
    h2 = jax.nn.relu(t2 * s2_ref[0] + b2_ref[0])
    out_ref[...] = lax.dot_general(
        h2, w3_ref[...], (((1,), (0,)), ((), ()))) + l3b_ref[0]


def _head(p1, p2, p3, p4, w1, l1b, s1, b1, w2, l2b, s2, b2, w3, l3b):
    full = lambda *s: pl.BlockSpec(s, lambda: tuple(0 for _ in s))
    return pl.pallas_call(
        _head_body,
        in_specs=[
            full(B, 1, 64), full(B, 1, 64), full(B, 1, 64), full(B, 1, 128),
            full(320, 1024), full(1, 1024), full(1, 1024), full(1, 1024),
            full(1024, 512), full(1, 512), full(1, 512), full(1, 512),
            full(512, 3), full(1, 3),
        ],
        out_specs=pl.BlockSpec((B, 3), lambda: (0, 0)),
        out_shape=jax.ShapeDtypeStruct((B, 3), jnp.float32),
    )(p1, p2, p3, p4, w1, l1b, s1, b1, w2, l2b, s2, b2, w3, l3b)


def _layer(xf, w, g, bc, cin, cpad, cout):
    xx = jnp.sum(xf * xf, axis=2)
    xp = xf
    if cpad != cin:
        xp = jnp.concatenate(
            [xf, jnp.zeros((B, N, cpad - cin), jnp.float32)], axis=2)
    idx = _knn(xp, xx)
    idx2d = idx.reshape(BN * K // 128, 128)
    diff = _make_sc_diff(cpad)(xp.reshape(BN, cpad), idx2d)
    zc = jnp.zeros((cpad - cin, cout), jnp.float32)
    wd = jnp.concatenate([w[:, :cin].T, zc], axis=0)
    wx = jnp.concatenate([w[:, cin:].T, zc], axis=0)
    s = (g / jnp.sqrt(1.0 + EPS))[None, :]
    xnext, hm = _conv(diff.reshape(B, N, K * cpad), xp, wd, wx, s,
                      bc[None, :], cout)
    return xnext, hm


def kernel(x, W1, W2, W3, W4, gc1, bc1, gc2, bc2, gc3, bc3, gc4, bc4,
           L1w, L1b, g1, b1, L2w, L2b, g2, b2, L3w, L3b):
    x1, hm1 = _layer(x, W1, gc1, bc1, 3, 16, 64)
    x2, hm2 = _layer(x1, W2, gc2, bc2, 64, 64, 64)
    x3, hm3 = _layer(x2, W3, gc3, bc3, 64, 64, 64)
    _, hm4 = _layer(x3, W4, gc4, bc4, 64, 64, 128)

    s1 = (g1 / jnp.sqrt(1.0 + EPS))[None, :]
    s2 = (g2 / jnp.sqrt(1.0 + EPS))[None, :]
    return _head(hm1, hm2, hm3, hm4,
                 L1w.T, L1b[None, :], s1, b1[None, :],
                 L2w.T, L2b[None, :], s2, b2[None, :],
                 L3w.T, L3b[None, :])


# speedup vs baseline: 1.0326x; 1.0326x over previous
import functools

import jax
import jax.numpy as jnp
from jax import lax
from jax.experimental import pallas as pl
from jax.experimental.pallas import tpu as pltpu
from jax.experimental.pallas import tpu_sc as plsc

B, N, K = 8, 2048, 20
BN = B * N
EPS = 1e-5
NEG = -3.0e38


def _knn_body(xt_ref, xa_ref, xxc_ref, xxr_ref, idx_ref, *, tn):
    b = pl.program_id(0)
    xt = xt_ref[0]
    xa = xa_ref[0]
    dot = lax.dot_general(xt, xa, (((1,), (1,)), ((), ())))
    pd = (2.0 * dot - xxc_ref[0]) - xxr_ref[0]
    base = b * N
    q = N // 4
    iota = lax.broadcasted_iota(jnp.int32, (tn, q), 1)
    vals = [pd[:, i * q:(i + 1) * q] for i in range(4)]
    idxs = [iota + i * q for i in range(4)]

    def comp(p, r):
        sw = vals[r] > vals[p]
        vp = jnp.where(sw, vals[r], vals[p])
        vr = jnp.where(sw, vals[p], vals[r])
        ip = jnp.where(sw, idxs[r], idxs[p])
        ir = jnp.where(sw, idxs[p], idxs[r])
        vals[p], vals[r], idxs[p], idxs[r] = vp, vr, ip, ir

    for p, r in ((0, 1), (1, 2), (2, 3), (0, 1), (1, 2), (0, 1)):
        comp(p, r)
    s0, s1, s2, s3 = vals
    i0, i1, i2, i3 = idxs
    for it in range(K):
        m = jnp.max(s0, axis=1, keepdims=True)
        cand = jnp.where(s0 == m, i0, N)
        j = jnp.min(cand, axis=1, keepdims=True)
        idx_ref[0, :, pl.ds(it, 1)] = j + base
        hit = i0 == j
        s0 = jnp.where(hit, s1, s0)
        i0 = jnp.where(hit, i1, i0)
        s1 = jnp.where(hit, s2, s1)
        i1 = jnp.where(hit, i2, i1)
        s2 = jnp.where(hit, s3, s2)
        i2 = jnp.where(hit, i3, i2)
        s3 = jnp.where(hit, NEG, s3)


def _knn(xp, xx, tn=256):
    cp = xp.shape[-1]
    nb = xp.shape[0]
    return pl.pallas_call(
        functools.partial(_knn_body, tn=tn),
        grid=(nb, N // tn),
        in_specs=[
            pl.BlockSpec((1, tn, cp), lambda b, t: (b, t, 0)),
            pl.BlockSpec((1, N, cp), lambda b, t: (b, 0, 0)),
            pl.BlockSpec((1, tn, 1), lambda b, t: (b, t, 0)),
            pl.BlockSpec((1, 1, N), lambda b, t: (b, 0, 0)),
        ],
        out_specs=pl.BlockSpec((1, tn, K), lambda b, t: (b, t, 0)),
        out_shape=jax.ShapeDtypeStruct((nb, N, K), jnp.int32),
    )(xp, xp, xx[:, :, None], xx[:, None, :])


_NC, _NS = 2, 16
_NW = _NC * _NS
_P = 32
_NIDX = _P * K // 128


@functools.lru_cache(maxsize=None)
def _make_sc_diff(c, bn):
    _PTS = bn // _NW
    _NCHUNK = _PTS // _P
    mesh = plsc.VectorSubcoreMesh(core_axis_name="c", subcore_axis_name="s")

    @functools.partial(
        pl.kernel,
        mesh=mesh,
        compiler_params=pltpu.CompilerParams(use_tc_tiling_on_sc=False),
        out_type=jax.ShapeDtypeStruct((bn * K, c), jnp.float32),
        scratch_types=[
            pltpu.VMEM((_PTS * K // 128, 128), jnp.int32),
            pltpu.VMEM((2, _P * K, c), jnp.float32),
            pltpu.VMEM((2, _P, c), jnp.float32),
            pltpu.SemaphoreType.DMA,
            pltpu.SemaphoreType.DMA,
            pltpu.SemaphoreType.DMA,
            pltpu.SemaphoreType.DMA,
            pltpu.SemaphoreType.DMA,
            pltpu.SemaphoreType.DMA,
        ],
    )
    def sc_diff(x_hbm, idx_hbm, diff_hbm, idx_v, rows_v, self_v,
                g0, g1, s0, s1, o0, o1):
        gsem = (g0, g1)
        ssem = (s0, s1)
        osem = (o0, o1)
        cid = lax.axis_index("c")
        sid = lax.axis_index("s")
        wid = cid * _NS + sid
        base = wid * _PTS
        nrow = _PTS * K // 128
        pltpu.sync_copy(idx_hbm.at[pl.ds(wid * nrow, nrow)], idx_v)

        def fire(ch, buf):
            for i in range(_NIDX):
                pltpu.async_copy(x_hbm.at[idx_v.at[ch * _NIDX + i]],
                                 rows_v.at[buf, pl.ds(i * 128, 128)],
                                 gsem[buf])
            pltpu.async_copy(x_hbm.at[pl.ds(base + ch * _P, _P)],
                             self_v.at[buf], ssem[buf])

        def drain(buf):
            for i in range(_NIDX):
                pltpu.make_async_copy(
                    x_hbm.at[pl.ds(0, 128)],
                    rows_v.at[buf, pl.ds(i * 128, 128)], gsem[buf]).wait()
            pltpu.make_async_copy(x_hbm.at[pl.ds(base, _P)],
                                  self_v.at[buf], ssem[buf]).wait()

        def wait_out(buf):
            pltpu.make_async_copy(diff_hbm.at[pl.ds(0, _P * K)],
                                  rows_v.at[buf], osem[buf]).wait()

        def step(ch, buf):
            @pl.when(ch + 1 < _NCHUNK)
            def _():
                @pl.when(ch >= 1)
                def _():
                    wait_out(1 - buf)
                fire(ch + 1, 1 - buf)

            drain(buf)

            def pt_body(p, carry2):
                r = p * K
                for cc in range(c // 16):
                    sl = pl.ds(cc * 16, 16)
                    s = self_v[buf, p, sl]
                    for kk in range(K):
                        rows_v[buf, r + kk, sl] = rows_v[buf, r + kk, sl] - s
                return carry2

            lax.fori_loop(0, _P, pt_body, 0)
            pbase = base + ch * _P
            pltpu.async_copy(rows_v.at[buf],
                             diff_hbm.at[pl.ds(pbase * K, _P * K)],
                             osem[buf])

        fire(0, 0)

        def pair_body(g, carry):
            step(2 * g, 0)
            step(2 * g + 1, 1)
            return carry

        lax.fori_loop(0, _NCHUNK // 2, pair_body, 0)
        wait_out(0)
        wait_out(1)

    return sc_diff


def _conv_body(diff_ref, x_ref, wd_ref, wx_ref, s_ref, b_ref,
               xn_ref, hm_ref, *, c):
    t = pl.program_id(1)
    dd = diff_ref[0]
    acc = lax.dot_general(dd[:, :c], wd_ref[...], (((1,), (0,)), ((), ())))
    for kk in range(1, K):
        acc = jnp.maximum(acc, lax.dot_general(
            dd[:, kk * c:(kk + 1) * c], wd_ref[...],
            (((1,), (0,)), ((), ()))))
    ys = lax.dot_general(x_ref[0], wx_ref[...], (((1,), (0,)), ((), ())))
    y = jnp.maximum(s_ref[0] * (acc + ys) + b_ref[0], 0.0)
    xn_ref[0] = y
    tm = jnp.max(y, axis=0, keepdims=True)

    @pl.when(t == 0)
    def _():
        hm_ref[0] = tm

    @pl.when(t != 0)
    def _():
        hm_ref[0] = jnp.maximum(hm_ref[0], tm)


def _conv(diff, xp, wd, wx, s, bias, o, tn=256):
    c = xp.shape[-1]
    nb = xp.shape[0]
    return pl.pallas_call(
        functools.partial(_conv_body, c=c),
        grid=(nb, N // tn),
        in_specs=[
            pl.BlockSpec((1, tn, K * c), lambda b, t: (b, t, 0)),
            pl.BlockSpec((1, tn, c), lambda b, t: (b, t, 0)),
            pl.BlockSpec((c, o), lambda b, t: (0, 0)),
            pl.BlockSpec((c, o), lambda b, t: (0, 0)),
            pl.BlockSpec((1, o), lambda b, t: (0, 0)),
            pl.BlockSpec((1, o), lambda b, t: (0, 0)),
        ],
        out_specs=[
            pl.BlockSpec((1, tn, o), lambda b, t: (b, t, 0)),
            pl.BlockSpec((1, 1, o), lambda b, t: (b, 0, 0)),
        ],
        out_shape=[
            jax.ShapeDtypeStruct((nb, N, o), jnp.float32),
            jax.ShapeDtypeStruct((nb, 1, o), jnp.float32),
        ],
    )(diff, xp, wd, wx, s, bias)


def _head_body(p1_ref, p2_ref, p3_ref, p4_ref, w1_ref, l1b_ref, s1_ref,
               b1_ref, w2_ref, l2b_ref, s2_ref, b2_ref, w3_ref, l3b_ref,
               out_ref):
    h = jnp.concatenate([p1_ref[:, 0], p2_ref[:, 0], p3_ref[:, 0],
                         p4_ref[:, 0]], axis=1)
    t1 = lax.dot_general(h, w1_ref[...], (((1,), (0,)), ((), ()))) + l1b_ref[0]
    h1 = jax.nn.relu(t1 * s1_ref[0] + b1_ref[0])
    t2 = lax.dot_general(h1, w2_ref[...], (((1,), (0,)), ((), ()))) \
        + l2b_ref[0]
    h2 = jax.nn.relu(t2 * s2_ref[0] + b2_ref[0])
    out_ref[...] = lax.dot_general(
        h2, w3_ref[...], (((1,), (0,)), ((), ()))) + l3b_ref[0]


def _head(p1, p2, p3, p4, w1, l1b, s1, b1, w2, l2b, s2, b2, w3, l3b):
    full = lambda *s: pl.BlockSpec(s, lambda: tuple(0 for _ in s))
    return pl.pallas_call(
        _head_body,
        in_specs=[
            full(B, 1, 64), full(B, 1, 64), full(B, 1, 64), full(B, 1, 128),
            full(320, 1024), full(1, 1024), full(1, 1024), full(1, 1024),
            full(1024, 512), full(1, 512), full(1, 512), full(1, 512),
            full(512, 3), full(1, 3),
        ],
        out_specs=pl.BlockSpec((B, 3), lambda: (0, 0)),
        out_shape=jax.ShapeDtypeStruct((B, 3), jnp.float32),
    )(p1, p2, p3, p4, w1, l1b, s1, b1, w2, l2b, s2, b2, w3, l3b)


def _layer(xf, w, g, bc, cin, cpad, cout):
    hb = B // 2
    bnh = hb * N
    xx = jnp.sum(xf * xf, axis=2)
    xp = xf
    if cpad != cin:
        xp = jnp.concatenate(
            [xf, jnp.zeros((B, N, cpad - cin), jnp.float32)], axis=2)
    zc = jnp.zeros((cpad - cin, cout), jnp.float32)
    wd = jnp.concatenate([w[:, :cin].T, zc], axis=0)
    wx = jnp.concatenate([w[:, cin:].T, zc], axis=0)
    s = (g / jnp.sqrt(1.0 + EPS))[None, :]
    halves = [(xp[:hb], xx[:hb]), (xp[hb:], xx[hb:])]
    idxs = [_knn(xph, xxh) for xph, xxh in halves]
    sc = _make_sc_diff(cpad, bnh)
    diffs = [sc(xph.reshape(bnh, cpad), idx.reshape(bnh * K // 128, 128))
             for (xph, _), idx in zip(halves, idxs)]
    outs = [_conv(d.reshape(hb, N, K * cpad), xph, wd, wx, s, bc[None, :],
                  cout)
            for d, (xph, _) in zip(diffs, halves)]
    xnext = jnp.concatenate([outs[0][0], outs[1][0]], axis=0)
    hm = jnp.concatenate([outs[0][1], outs[1][1]], axis=0)
    return xnext, hm


def kernel(x, W1, W2, W3, W4, gc1, bc1, gc2, bc2, gc3, bc3, gc4, bc4,
           L1w, L1b, g1, b1, L2w, L2b, g2, b2, L3w, L3b):
    x1, hm1 = _layer(x, W1, gc1, bc1, 3, 16, 64)
    x2, hm2 = _layer(x1, W2, gc2, bc2, 64, 64, 64)
    x3, hm3 = _layer(x2, W3, gc3, bc3, 64, 64, 64)
    _, hm4 = _layer(x3, W4, gc4, bc4, 64, 64, 128)

    s1 = (g1 / jnp.sqrt(1.0 + EPS))[None, :]
    s2 = (g2 / jnp.sqrt(1.0 + EPS))[None, :]
    return _head(hm1, hm2, hm3, hm4,
                 L1w.T, L1b[None, :], s1, b1[None, :],
                 L2w.T, L2b[None, :], s2, b2[None, :],
                 L3w.T, L3b[None, :])
